# Initial kernel scaffold; baseline (speedup 1.0000x reference)
import jax
import jax.numpy as jnp
from jax import lax
from jax.experimental import pallas as pl
from jax.experimental.pallas import tpu as pltpu
from jax.experimental.pallas import tpu_sc as plsc

B, S = 16384, 50
D = 32
N_ROWS = B * S
NC, NS = 2, 16
NW = NC * NS
PER_W = N_ROWS // NW
CHUNK = 1024
N_CHUNKS = PER_W // CHUNK


def _gather_body(idx_hbm, table_hbm, out_hbm, idx_v, rows_v, sem):
    wid = lax.axis_index("s") * NC + lax.axis_index("c")
    base = wid * PER_W

    def body(g, carry):
        off = base + g * CHUNK
        pltpu.sync_copy(idx_hbm.at[pl.ds(off, CHUNK)], idx_v)
        pltpu.async_copy(table_hbm.at[idx_v], rows_v, sem).wait()
        pltpu.sync_copy(rows_v, out_hbm.at[pl.ds(off, CHUNK)])
        return carry

    lax.fori_loop(0, N_CHUNKS, body, 0)


def kernel(xy_id, emb_table):
    flat_idx = xy_id.reshape(N_ROWS)
    mesh = plsc.VectorSubcoreMesh(core_axis_name="c", subcore_axis_name="s")
    run = pl.kernel(
        _gather_body,
        mesh=mesh,
        out_type=jax.ShapeDtypeStruct((N_ROWS, D), jnp.float32),
        scratch_types=[
            pltpu.VMEM((CHUNK,), jnp.int32),
            pltpu.VMEM((CHUNK, D), jnp.float32),
            pltpu.SemaphoreType.DMA,
        ],
    )
    out = run(flat_idx, emb_table)
    return out.reshape(B, S, D)


# speedup vs baseline: 1.0949x; 1.0949x over previous
import jax
import jax.numpy as jnp
from jax import lax
from jax.experimental import pallas as pl
from jax.experimental.pallas import tpu as pltpu
from jax.experimental.pallas import tpu_sc as plsc

B, S = 16384, 50
D = 32
N_ROWS = B * S
NC, NS = 2, 16
NW = NC * NS
PER_W = N_ROWS // NW
CHUNK = 1024
N_CHUNKS = PER_W // CHUNK


def _gather_body(idx_hbm, table_hbm, out_hbm, idx_v, rows_v, sem):
    wid = lax.axis_index("s") * NC + lax.axis_index("c")
    base = wid * PER_W

    def body(g, carry):
        off = base + g * CHUNK
        pltpu.sync_copy(idx_hbm.at[pl.ds(off, CHUNK)], idx_v)
        pltpu.async_copy(table_hbm.at[idx_v], rows_v, sem).wait()
        pltpu.sync_copy(rows_v, out_hbm.at[pl.ds(off, CHUNK)])
        return carry

    lax.fori_loop(0, N_CHUNKS, body, 0)


def kernel(xy_id, emb_table):
    flat_idx = xy_id.reshape(N_ROWS)
    mesh = plsc.VectorSubcoreMesh(core_axis_name="c", subcore_axis_name="s")
    run = pl.kernel(
        _gather_body,
        mesh=mesh,
        out_type=jax.ShapeDtypeStruct((N_ROWS, D), jnp.float32),
        scratch_types=[
            pltpu.VMEM((CHUNK,), jnp.int32),
            pltpu.VMEM((CHUNK, D), jnp.float32),
            pltpu.SemaphoreType.DMA,
        ],
        compiler_params=pltpu.CompilerParams(use_tc_tiling_on_sc=False),
    )
    out = run(flat_idx, emb_table)
    return out.reshape(B, S, D)


# speedup vs baseline: 1.1150x; 1.0184x over previous
import jax
import jax.numpy as jnp
from jax import lax
from jax.experimental import pallas as pl
from jax.experimental.pallas import tpu as pltpu
from jax.experimental.pallas import tpu_sc as plsc

B, S = 16384, 50
D = 32
N_ROWS = B * S
NC, NS = 2, 16
NW = NC * NS
PER_W = N_ROWS // NW
CHUNK = 640
N_CHUNKS = PER_W // CHUNK
NBUF = 4


def _gather_body(idx_hbm, table_hbm, out_hbm,
                 idx_all, r0, r1, r2, r3,
                 g0, g1, g2, g3, s0, s1, s2, s3):
    rows = (r0, r1, r2, r3)
    gsem = (g0, g1, g2, g3)
    ssem = (s0, s1, s2, s3)
    wid = lax.axis_index("s") * NC + lax.axis_index("c")
    base = wid * PER_W

    pltpu.sync_copy(idx_hbm.at[pl.ds(base, PER_W)], idx_all)

    def gather_desc(b, c):
        return pltpu.make_async_copy(
            table_hbm.at[idx_all.at[pl.ds(c * CHUNK, CHUNK)]], rows[b], gsem[b])

    def store_desc(b, c):
        return pltpu.make_async_copy(
            rows[b], out_hbm.at[pl.ds(base + c * CHUNK, CHUNK)], ssem[b])

    for b in range(NBUF):
        gather_desc(b, b).start()

    def group(G, carry):
        for b in range(NBUF):
            c = G * NBUF + b
            gather_desc(b, c).wait()
            store_desc(b, c).start()
            store_desc(b, c).wait()
            gather_desc(b, c + NBUF).start()
        return carry

    lax.fori_loop(0, (N_CHUNKS - NBUF) // NBUF, group, 0)

    for b in range(NBUF):
        c = N_CHUNKS - NBUF + b
        gather_desc(b, c).wait()
        store_desc(b, c).start()
    for b in range(NBUF):
        c = N_CHUNKS - NBUF + b
        store_desc(b, c).wait()


def kernel(xy_id, emb_table):
    flat_idx = xy_id.reshape(N_ROWS)
    mesh = plsc.VectorSubcoreMesh(core_axis_name="c", subcore_axis_name="s")
    run = pl.kernel(
        _gather_body,
        mesh=mesh,
        out_type=jax.ShapeDtypeStruct((N_ROWS, D), jnp.float32),
        scratch_types=[
            pltpu.VMEM((PER_W,), jnp.int32),
        ] + [pltpu.VMEM((CHUNK, D), jnp.float32)] * NBUF
          + [pltpu.SemaphoreType.DMA] * (2 * NBUF),
        compiler_params=pltpu.CompilerParams(use_tc_tiling_on_sc=False),
    )
    out = run(flat_idx, emb_table)
    return out.reshape(B, S, D)


# speedup vs baseline: 1.6380x; 1.4691x over previous
import jax
import jax.numpy as jnp
from jax import lax
from jax.experimental import pallas as pl
from jax.experimental.pallas import tpu as pltpu
from jax.experimental.pallas import tpu_sc as plsc

B, S = 16384, 50
D = 32
NC, NS = 2, 16
NW = NC * NS
CW = B // NW
TCW = CW // 128
L = 16


def _body(idx_hbm, table_hbm, out_hbm, idx_v, r0, r1, st0, st1,
          g0, g1, s0, s1):
    rows = (r0, r1)
    stage = (st0, st1)
    gsem = (g0, g1)
    ssem = (s0, s1)
    wid = lax.axis_index("s") * NC + lax.axis_index("c")
    b0 = wid * CW

    pltpu.sync_copy(idx_hbm.at[:, pl.ds(b0, CW)], idx_v)

    def gather_desc(bf, s):
        return pltpu.make_async_copy(
            table_hbm.at[idx_v.at[s]], rows[bf], gsem[bf])

    def store_desc(bf, s):
        return pltpu.make_async_copy(
            stage[bf],
            out_hbm.at[pl.ds(s * 4, 4), pl.ds(wid * TCW, TCW), :, :],
            ssem[bf])

    iota = lax.iota(jnp.int32, L)

    def transpose(bf):
        rbuf = rows[bf]
        sbuf = stage[bf]

        def lb_body(lb, carry):
            col16 = iota * 0
            for trl in range(4):
                for r in range(8):
                    col = col16 + (8 * trl + r)
                    for tcl in range(4):
                        row = iota + (128 * tcl) + lb * L
                        v = plsc.load_gather(rbuf, [row, col])
                        sbuf[trl, tcl, r, pl.ds(lb * L, L)] = v
            return carry

        lax.fori_loop(0, 128 // L, lb_body, 0)

    gather_desc(0, 0).start()
    gather_desc(1, 1).start()
    for s in (0, 1):
        bf = s & 1
        gather_desc(bf, s).wait()
        transpose(bf)
        store_desc(bf, s).start()
        gather_desc(bf, s + 2).start()

    def main_body(g, carry):
        s_pair = 2 + 2 * g
        for bf in (0, 1):
            s = s_pair + bf
            gather_desc(bf, s).wait()
            store_desc(bf, s - 2).wait()
            transpose(bf)
            store_desc(bf, s).start()
            gather_desc(bf, s + 2).start()
        return carry

    lax.fori_loop(0, (S - 4) // 2, main_body, 0)

    for s in (S - 2, S - 1):
        bf = s & 1
        gather_desc(bf, s).wait()
        store_desc(bf, s - 2).wait()
        transpose(bf)
        store_desc(bf, s).start()
    store_desc(0, S - 2).wait()
    store_desc(1, S - 1).wait()


def kernel(xy_id, emb_table):
    idx2d = xy_id.T
    mesh = plsc.VectorSubcoreMesh(core_axis_name="c", subcore_axis_name="s")
    run = pl.kernel(
        _body,
        mesh=mesh,
        out_type=jax.ShapeDtypeStruct((200, 128, 8, 128), jnp.float32),
        scratch_types=[
            pltpu.VMEM((S, CW), jnp.int32),
            pltpu.VMEM((CW, D), jnp.float32),
            pltpu.VMEM((CW, D), jnp.float32),
            pltpu.VMEM((4, TCW, 8, 128), jnp.float32),
            pltpu.VMEM((4, TCW, 8, 128), jnp.float32),
            pltpu.SemaphoreType.DMA,
            pltpu.SemaphoreType.DMA,
            pltpu.SemaphoreType.DMA,
            pltpu.SemaphoreType.DMA,
        ],
        compiler_params=pltpu.CompilerParams(
            use_tc_tiling_on_sc=False, needs_layout_passes=False),
    )
    out4 = run(idx2d, emb_table.reshape(1000000, D))
    return out4.transpose(0, 2, 1, 3).reshape(S, D, B).transpose(2, 0, 1)


# speedup vs baseline: 2.6033x; 1.5894x over previous
import jax
import jax.numpy as jnp
from jax import lax
from jax.experimental import pallas as pl
from jax.experimental.pallas import tpu as pltpu
from jax.experimental.pallas import tpu_sc as plsc

B, S = 16384, 50
D = 32
NC, NS = 2, 16
NW = NC * NS
CW = B // NW
TCW = CW // 128
L = 16


def _body(idx_hbm, table_hbm, out_hbm, idx_v, r0, r1, st0, st1,
          g0, g1, s0, s1):
    rows = (r0, r1)
    stage = (st0, st1)
    gsem = (g0, g1)
    ssem = (s0, s1)
    wid = lax.axis_index("s") * NC + lax.axis_index("c")
    b0 = wid * CW

    pltpu.sync_copy(idx_hbm.at[:, pl.ds(b0, CW)], idx_v)

    def gather_desc(bf, s):
        return pltpu.make_async_copy(
            table_hbm.at[idx_v.at[s]], rows[bf], gsem[bf])

    def store_desc(bf, s):
        return pltpu.make_async_copy(
            stage[bf],
            out_hbm.at[pl.ds(s * 4, 4), pl.ds(wid * TCW, TCW), :, :],
            ssem[bf])

    iota = lax.iota(jnp.int32, L)
    zero = iota * 0

    def transpose(bf):
        rbuf = rows[bf]
        sbuf = stage[bf]

        def j_body(jb, carry):
            row = iota + jb * L
            tcl_v = zero + (jb // 8)
            l_v = iota + (jb & 7) * L
            for d0 in range(D):
                col = (iota + d0) & (D - 1)
                v = plsc.load_gather(rbuf, [row, col])
                plsc.store_scatter(
                    sbuf, [col >> 3, tcl_v, col & 7, l_v], v)
            return carry

        lax.fori_loop(0, CW // L, j_body, 0)

    gather_desc(0, 0).start()
    gather_desc(1, 1).start()
    for s in (0, 1):
        bf = s & 1
        gather_desc(bf, s).wait()
        transpose(bf)
        store_desc(bf, s).start()
        gather_desc(bf, s + 2).start()

    def main_body(g, carry):
        s_pair = 2 + 2 * g
        for bf in (0, 1):
            s = s_pair + bf
            gather_desc(bf, s).wait()
            store_desc(bf, s - 2).wait()
            transpose(bf)
            store_desc(bf, s).start()
            gather_desc(bf, s + 2).start()
        return carry

    lax.fori_loop(0, (S - 4) // 2, main_body, 0)

    for s in (S - 2, S - 1):
        bf = s & 1
        gather_desc(bf, s).wait()
        store_desc(bf, s - 2).wait()
        transpose(bf)
        store_desc(bf, s).start()
    store_desc(0, S - 2).wait()
    store_desc(1, S - 1).wait()


def kernel(xy_id, emb_table):
    idx2d = xy_id.T
    mesh = plsc.VectorSubcoreMesh(core_axis_name="c", subcore_axis_name="s")
    run = pl.kernel(
        _body,
        mesh=mesh,
        out_type=jax.ShapeDtypeStruct((200, 128, 8, 128), jnp.float32),
        scratch_types=[
            pltpu.VMEM((S, CW), jnp.int32),
            pltpu.VMEM((CW, D), jnp.float32),
            pltpu.VMEM((CW, D), jnp.float32),
            pltpu.VMEM((4, TCW, 8, 128), jnp.float32),
            pltpu.VMEM((4, TCW, 8, 128), jnp.float32),
            pltpu.SemaphoreType.DMA,
            pltpu.SemaphoreType.DMA,
            pltpu.SemaphoreType.DMA,
            pltpu.SemaphoreType.DMA,
        ],
        compiler_params=pltpu.CompilerParams(
            use_tc_tiling_on_sc=False, needs_layout_passes=False),
    )
    out4 = run(idx2d, emb_table.reshape(1000000, D))
    return out4.transpose(0, 2, 1, 3).reshape(S, D, B).transpose(2, 0, 1)


# speedup vs baseline: 2.6349x; 1.0121x over previous
import jax
import jax.numpy as jnp
from jax import lax
from jax.experimental import pallas as pl
from jax.experimental.pallas import tpu as pltpu
from jax.experimental.pallas import tpu_sc as plsc

B, S = 16384, 50
D = 32
NC, NS = 2, 16
NW = NC * NS
CW = B // NW
TCW = CW // 128
L = 16


def _body(idx_hbm, table_hbm, out_hbm, idx_v, r0, r1, st0, st1,
          g0, g1, s0, s1):
    rows = (r0, r1)
    stage = (st0, st1)
    gsem = (g0, g1)
    ssem = (s0, s1)
    wid = lax.axis_index("s") * NC + lax.axis_index("c")
    b0 = wid * CW

    pltpu.sync_copy(idx_hbm.at[:, pl.ds(b0, CW)], idx_v)

    def gather_desc(bf, s):
        return pltpu.make_async_copy(
            table_hbm.at[idx_v.at[s]], rows[bf], gsem[bf])

    def store_desc(bf, s):
        return pltpu.make_async_copy(
            stage[bf],
            out_hbm.at[pl.ds(s * 4, 4), pl.ds(wid * TCW, TCW), :, :],
            ssem[bf])

    iota = lax.iota(jnp.int32, L)
    zero = iota * 0

    def transpose(bf):
        rbuf = rows[bf]
        sbuf = stage[bf]

        def j_body(jb, carry):
            row = iota + jb * L
            tcl_v = zero + (jb // 8)
            l_v = iota + (jb & 7) * L
            for d0 in range(D):
                col = iota ^ d0
                v = plsc.load_gather(rbuf, [row, col])
                plsc.store_scatter(
                    sbuf, [col >> 3, tcl_v, col & 7, l_v], v)
            return carry

        lax.fori_loop(0, CW // L, j_body, 0)

    gather_desc(0, 0).start()
    gather_desc(1, 1).start()
    for s in (0, 1):
        bf = s & 1
        gather_desc(bf, s).wait()
        transpose(bf)
        store_desc(bf, s).start()
        gather_desc(bf, s + 2).start()

    def main_body(g, carry):
        s_pair = 2 + 2 * g
        for bf in (0, 1):
            s = s_pair + bf
            gather_desc(bf, s).wait()
            store_desc(bf, s - 2).wait()
            transpose(bf)
            store_desc(bf, s).start()
            gather_desc(bf, s + 2).start()
        return carry

    lax.fori_loop(0, (S - 4) // 2, main_body, 0)

    for s in (S - 2, S - 1):
        bf = s & 1
        gather_desc(bf, s).wait()
        store_desc(bf, s - 2).wait()
        transpose(bf)
        store_desc(bf, s).start()
    store_desc(0, S - 2).wait()
    store_desc(1, S - 1).wait()


def kernel(xy_id, emb_table):
    idx2d = xy_id.T
    mesh = plsc.VectorSubcoreMesh(core_axis_name="c", subcore_axis_name="s")
    run = pl.kernel(
        _body,
        mesh=mesh,
        out_type=jax.ShapeDtypeStruct((200, 128, 8, 128), jnp.float32),
        scratch_types=[
            pltpu.VMEM((S, CW), jnp.int32),
            pltpu.VMEM((CW, D), jnp.float32),
            pltpu.VMEM((CW, D), jnp.float32),
            pltpu.VMEM((4, TCW, 8, 128), jnp.float32),
            pltpu.VMEM((4, TCW, 8, 128), jnp.float32),
            pltpu.SemaphoreType.DMA,
            pltpu.SemaphoreType.DMA,
            pltpu.SemaphoreType.DMA,
            pltpu.SemaphoreType.DMA,
        ],
        compiler_params=pltpu.CompilerParams(
            use_tc_tiling_on_sc=False, needs_layout_passes=False),
    )
    out4 = run(idx2d, emb_table.reshape(1000000, D))
    return out4.transpose(0, 2, 1, 3).reshape(S, D, B).transpose(2, 0, 1)
